# baseline (speedup 1.0000x reference)
import functools

import jax
import jax.numpy as jnp
from jax.experimental import pallas as pl
from jax.experimental.pallas import tpu as pltpu

F32 = jnp.float32
_EPS = 1e-5
_N = 1024
_NT = 128


def _convT(x, wmat, bias):
    a, c, b, n = x.shape
    za = jnp.zeros((1, c, b, n), x.dtype)
    xp = jnp.concatenate([za, x, za], axis=0)
    zb = jnp.zeros((a + 2, c, 1, n), x.dtype)
    xp = jnp.concatenate([zb, xp, zb], axis=2)
    patch = jnp.concatenate(
        [xp[da:da + a, :, db:db + b, :] for da in range(3) for db in range(3)],
        axis=1)
    y = jax.lax.dot_general(wmat, patch, (((1,), (1,)), ((), ())),
                            preferred_element_type=F32)
    return y + bias[:, :, None, None]


def _poolT(x, sel):
    c, a, b, n = x.shape
    x5 = x.reshape(c, a // 2, 2, b, n)
    t = jnp.maximum(x5[:, :, 0], x5[:, :, 1])
    m = jnp.maximum(t[:, :, 0:b - 1], t[:, :, 1:b])
    return jax.lax.dot_general(sel, m, (((1,), (2,)), ((), ())),
                               preferred_element_type=F32)


def _accum_stats(st_ref, y):
    i = pl.program_id(0)
    axes = tuple(range(1, y.ndim))
    ps = jnp.sum(y, axis=axes)[:, None]
    pq = jnp.sum(y * y, axis=axes)[:, None]
    upd = jnp.concatenate([ps, pq], axis=1)

    @pl.when(i == 0)
    def _():
        st_ref[...] = jnp.zeros_like(st_ref)

    st_ref[...] += upd


def _bn_coeffs(st, n, g, b):
    m = st[:, 0:1] / n
    v = st[:, 1:2] / n - m * m
    sc = g * jax.lax.rsqrt(v + _EPS)
    return sc, b - m * sc


def _bn_relu_pool(y, st, n, g, b, sel):
    sc, sh = _bn_coeffs(st, n, g, b)
    x = jnp.maximum(y * sc[:, :, None, None] + sh[:, :, None, None], 0.0)
    return _poolT(x, sel)


def _conv1_kernel(x_ref, w_ref, b_ref, st_ref):
    y = _convT(x_ref[...], w_ref[...], b_ref[...])
    _accum_stats(st_ref, y)


def _conv2_kernel(x_ref, st1_ref, g1_ref, b1_ref, sel1_ref, w1_ref, c1_ref,
                  w2_ref, c2_ref, st_ref):
    y1 = _convT(x_ref[...], w1_ref[...], c1_ref[...])
    x2 = _bn_relu_pool(y1, st1_ref[...], float(_N * 32 * 32), g1_ref[...],
                       b1_ref[...], sel1_ref[...])
    y2 = _convT(x2, w2_ref[...], c2_ref[...])
    _accum_stats(st_ref, y2)


def _conv3_kernel(x_ref, st1_ref, g1_ref, b1_ref, sel1_ref, w1_ref, c1_ref,
                  st2_ref, g2_ref, b2_ref, sel2_ref, w2_ref, c2_ref,
                  w3_ref, c3_ref, y_ref, st_ref):
    y1 = _convT(x_ref[...], w1_ref[...], c1_ref[...])
    x2 = _bn_relu_pool(y1, st1_ref[...], float(_N * 32 * 32), g1_ref[...],
                       b1_ref[...], sel1_ref[...])
    y2 = _convT(x2, w2_ref[...], c2_ref[...])
    x3 = _bn_relu_pool(y2, st2_ref[...], float(_N * 16 * 16), g2_ref[...],
                       b2_ref[...], sel2_ref[...])
    y3 = _convT(x3, w3_ref[...], c3_ref[...])
    _accum_stats(st_ref, y3)
    y_ref[...] = y3


def _fc_kernel(x_ref, sti_ref, g3_ref, b3_ref, fw_ref, fb_ref, lg_ref, lb_ref,
               dw_ref, db_ref, feat_ref, dlin_ref, st_ref):
    n3 = float(_N * 8 * 8)
    sc, sh = _bn_coeffs(sti_ref[...], n3, g3_ref[...], b3_ref[...])
    x = jnp.maximum(x_ref[...] * sc[:, :, None, None] + sh[:, :, None, None],
                    0.0)
    x = x.reshape(4096, _NT)
    f0 = jnp.dot(fw_ref[...], x, preferred_element_type=F32) + fb_ref[...]
    mu = jnp.mean(f0, axis=0, keepdims=True)
    va = jnp.mean(f0 * f0, axis=0, keepdims=True) - mu * mu
    feat = (f0 - mu) * jax.lax.rsqrt(va + _EPS) * lg_ref[...] + lb_ref[...]
    feat = jnp.maximum(feat, 0.0)
    dlin = jnp.dot(dw_ref[...], feat, preferred_element_type=F32) + db_ref[...]
    feat_ref[...] = feat
    dlin_ref[...] = dlin
    _accum_stats(st_ref, dlin)


def _head_kernel(feat_ref, dlin_ref, st_ref, gum_ref, bg_ref, bb_ref,
                 dfw_ref, dfb_ref, sww_ref, swb_ref, w1_ref, b1_ref,
                 w2_ref, b2_ref, cls_ref, dom_ref, idx_ref, hard_ref):
    sc, sh = _bn_coeffs(st_ref[...], float(_N), bg_ref[...], bb_ref[...])
    dp = jnp.maximum(dlin_ref[...] * sc + sh, 0.0)
    dom_ref[...] = (jnp.dot(dfw_ref[...], dp, preferred_element_type=F32)
                    + dfb_ref[...])
    s = jnp.dot(sww_ref[...], dp, preferred_element_type=F32) + swb_ref[...]
    s = s + gum_ref[...]
    ii = jax.lax.broadcasted_iota(jnp.int32, s.shape, 0)
    mx = jnp.max(s, axis=0, keepdims=True)
    idx = jnp.min(jnp.where(s == mx, ii, s.shape[0]), axis=0, keepdims=True)
    hard = (ii == idx).astype(F32)
    idx_ref[...] = idx
    hard_ref[...] = hard
    feat = feat_ref[...]
    acc = jnp.zeros((w2_ref.shape[1], feat.shape[1]), F32)
    for p in range(4):
        h = (jnp.dot(w1_ref[p], feat, preferred_element_type=F32)
             + b1_ref[p][:, None])
        h = jnp.maximum(h, 0.0)
        o = (jnp.dot(w2_ref[p], h, preferred_element_type=F32)
             + b2_ref[p][:, None])
        acc = acc + hard[p:p + 1, :] * o
    cls_ref[...] = acc


def _full(shape):
    nd = len(shape)
    return pl.BlockSpec(shape, lambda i: (0,) * nd)


_PARAMS = pltpu.CompilerParams(dimension_semantics=("arbitrary",))


def _sel(k):
    return (jnp.arange(2 * k - 1)[None, :] == 2 * jnp.arange(k)[:, None]
            ).astype(F32)


def kernel(input_data, conv1_w, conv1_b, bn1_g, bn1_b, conv2_w, conv2_b,
           bn2_g, bn2_b, conv3_w, conv3_b, bn3_g, bn3_b, fc1_w, fc1_b,
           ln_g, ln_b, disc_w, disc_b, bnd_g, bnd_b, discfc_w, discfc_b,
           sw_w, sw_b, part_w1, part_b1, part_w2, part_b2):
    x = input_data.transpose(2, 1, 3, 0)
    w1m = conv1_w.transpose(0, 2, 3, 1).reshape(16, 27)
    w2m = conv2_w.transpose(0, 3, 2, 1).reshape(32, 144)
    w3m = conv3_w.transpose(0, 2, 3, 1).reshape(64, 288)
    col = lambda a: a.reshape(-1, 1)

    g = _N // _NT
    xspec = pl.BlockSpec((32, 3, 32, _NT), lambda i: (0, 0, 0, i))
    st1 = pl.pallas_call(
        _conv1_kernel,
        grid=(g,),
        in_specs=[xspec, _full((16, 27)), _full((16, 1))],
        out_specs=pl.BlockSpec((16, 2), lambda i: (0, 0)),
        out_shape=jax.ShapeDtypeStruct((16, 2), F32),
        compiler_params=_PARAMS,
    )(x, w1m, col(conv1_b))

    st2 = pl.pallas_call(
        _conv2_kernel,
        grid=(g,),
        in_specs=[xspec, _full((16, 2)), _full((16, 1)), _full((16, 1)),
                  _full((16, 31)), _full((16, 27)), _full((16, 1)),
                  _full((32, 144)), _full((32, 1))],
        out_specs=pl.BlockSpec((32, 2), lambda i: (0, 0)),
        out_shape=jax.ShapeDtypeStruct((32, 2), F32),
        compiler_params=_PARAMS,
    )(x, st1, col(bn1_g), col(bn1_b), _sel(16), w1m, col(conv1_b),
      w2m, col(conv2_b))

    y3, st3 = pl.pallas_call(
        _conv3_kernel,
        grid=(g,),
        in_specs=[xspec, _full((16, 2)), _full((16, 1)), _full((16, 1)),
                  _full((16, 31)), _full((16, 27)), _full((16, 1)),
                  _full((32, 2)), _full((32, 1)), _full((32, 1)),
                  _full((8, 15)), _full((32, 144)), _full((32, 1)),
                  _full((64, 288)), _full((64, 1))],
        out_specs=[pl.BlockSpec((64, 8, 8, _NT), lambda i: (0, 0, 0, i)),
                   pl.BlockSpec((64, 2), lambda i: (0, 0))],
        out_shape=[jax.ShapeDtypeStruct((64, 8, 8, _N), F32),
                   jax.ShapeDtypeStruct((64, 2), F32)],
        compiler_params=_PARAMS,
    )(x, st1, col(bn1_g), col(bn1_b), _sel(16), w1m, col(conv1_b),
      st2, col(bn2_g), col(bn2_b), _sel(8), w2m, col(conv2_b),
      w3m, col(conv3_b))

    feat, dlin, std = pl.pallas_call(
        _fc_kernel,
        grid=(g,),
        in_specs=[pl.BlockSpec((64, 8, 8, _NT), lambda i: (0, 0, 0, i)),
                  _full((64, 2)), _full((64, 1)), _full((64, 1)),
                  _full((512, 4096)), _full((512, 1)),
                  _full((512, 1)), _full((512, 1)),
                  _full((512, 512)), _full((512, 1))],
        out_specs=[pl.BlockSpec((512, _NT), lambda i: (0, i)),
                   pl.BlockSpec((512, _NT), lambda i: (0, i)),
                   pl.BlockSpec((512, 2), lambda i: (0, 0))],
        out_shape=[jax.ShapeDtypeStruct((512, _N), F32),
                   jax.ShapeDtypeStruct((512, _N), F32),
                   jax.ShapeDtypeStruct((512, 2), F32)],
        compiler_params=_PARAMS,
    )(y3, st3, col(bn3_g), col(bn3_b), fc1_w, col(fc1_b), col(ln_g),
      col(ln_b), disc_w, col(disc_b))

    u = jax.random.uniform(jax.random.key(1234), (_N, 4), minval=1e-6,
                           maxval=1.0 - 1e-6)
    gumbel = (-jnp.log(-jnp.log(u))).T

    cls, dom, idx, hard = pl.pallas_call(
        _head_kernel,
        grid=(g,),
        in_specs=[pl.BlockSpec((512, _NT), lambda i: (0, i)),
                  pl.BlockSpec((512, _NT), lambda i: (0, i)),
                  _full((512, 2)),
                  pl.BlockSpec((4, _NT), lambda i: (0, i)),
                  _full((512, 1)), _full((512, 1)),
                  _full((2, 512)), _full((2, 1)),
                  _full((4, 512)), _full((4, 1)),
                  _full((4, 128, 512)), _full((4, 128)),
                  _full((4, 1000, 128)), _full((4, 1000))],
        out_specs=[pl.BlockSpec((1000, _NT), lambda i: (0, i)),
                   pl.BlockSpec((2, _NT), lambda i: (0, i)),
                   pl.BlockSpec((1, _NT), lambda i: (0, i)),
                   pl.BlockSpec((4, _NT), lambda i: (0, i))],
        out_shape=[jax.ShapeDtypeStruct((1000, _N), F32),
                   jax.ShapeDtypeStruct((2, _N), F32),
                   jax.ShapeDtypeStruct((1, _N), jnp.int32),
                   jax.ShapeDtypeStruct((4, _N), F32)],
        compiler_params=_PARAMS,
    )(feat, dlin, std, gumbel, col(bnd_g), col(bnd_b), discfc_w,
      col(discfc_b), sw_w, col(sw_b), part_w1, part_b1, part_w2, part_b2)

    return cls.T, dom.T, idx.reshape(_N), hard.T


# speedup vs baseline: 1.5362x; 1.5362x over previous
import jax
import jax.numpy as jnp
from jax.experimental import pallas as pl
from jax.experimental.pallas import tpu as pltpu

F32 = jnp.float32
_N = 1024
_NT = 128


def _head_kernel(feat_ref, dp_ref, sg_ref, dfw_ref, dfb_ref, w1_ref, b1_ref,
                 w2_ref, b2t_ref, cls_ref, dom_ref, idx_ref, hard_ref):
    dp = dp_ref[...]
    dom_ref[...] = (jax.lax.dot_general(dfw_ref[...], dp,
                                        (((1,), (0,)), ((), ())),
                                        preferred_element_type=F32)
                    + dfb_ref[...])
    s = sg_ref[...]
    ii = jax.lax.broadcasted_iota(jnp.int32, s.shape, 0)
    mx = jnp.max(s, axis=0, keepdims=True)
    idx = jnp.min(jnp.where(s == mx, ii, s.shape[0]), axis=0, keepdims=True)
    hard = (ii == idx).astype(F32)
    idx_ref[...] = idx
    hard_ref[...] = hard
    h = jax.lax.dot_general(w1_ref[...], feat_ref[...],
                            (((1,), (0,)), ((), ())),
                            preferred_element_type=F32) + b1_ref[...]
    h = jnp.maximum(h, 0.0)
    mask = jnp.concatenate(
        [jnp.broadcast_to(hard[p:p + 1], (128, h.shape[1])) for p in range(4)],
        axis=0)
    h = h * mask
    b2 = jax.lax.dot_general(b2t_ref[...], hard, (((1,), (0,)), ((), ())),
                             preferred_element_type=F32)
    cls_ref[...] = jax.lax.dot_general(w2_ref[...], h,
                                       (((1,), (0,)), ((), ())),
                                       preferred_element_type=F32) + b2


def _full(shape):
    nd = len(shape)
    return pl.BlockSpec(shape, lambda i: (0,) * nd)


_PARAMS = pltpu.CompilerParams(dimension_semantics=("arbitrary",))


def _bn2d(x, g, b, eps=1e-5):
    m = jnp.mean(x, axis=(0, 2, 3), keepdims=True)
    v = jnp.var(x, axis=(0, 2, 3), keepdims=True)
    return (x - m) / jnp.sqrt(v + eps) * g[None, :, None, None] + b[None, :, None, None]


def _bn1d(x, g, b, eps=1e-5):
    m = jnp.mean(x, axis=0, keepdims=True)
    v = jnp.var(x, axis=0, keepdims=True)
    return (x - m) / jnp.sqrt(v + eps) * g[None, :] + b[None, :]


def _ln(x, g, b, eps=1e-5):
    m = jnp.mean(x, axis=-1, keepdims=True)
    v = jnp.var(x, axis=-1, keepdims=True)
    return (x - m) / jnp.sqrt(v + eps) * g + b


def _conv(x, w, b):
    y = jax.lax.conv_general_dilated(x, w, window_strides=(1, 1),
                                     padding='SAME',
                                     dimension_numbers=('NCHW', 'OIHW', 'NCHW'))
    return y + b[None, :, None, None]


def _maxpool2(x):
    return jax.lax.reduce_window(x, -jnp.inf, jax.lax.max, (1, 1, 2, 2),
                                 (1, 1, 2, 2), 'VALID')


def kernel(input_data, conv1_w, conv1_b, bn1_g, bn1_b, conv2_w, conv2_b,
           bn2_g, bn2_b, conv3_w, conv3_b, bn3_g, bn3_b, fc1_w, fc1_b,
           ln_g, ln_b, disc_w, disc_b, bnd_g, bnd_b, discfc_w, discfc_b,
           sw_w, sw_b, part_w1, part_b1, part_w2, part_b2):
    x = jax.nn.relu(_bn2d(_conv(input_data, conv1_w, conv1_b), bn1_g, bn1_b))
    x = _maxpool2(x)
    x = jax.nn.relu(_bn2d(_conv(x, conv2_w, conv2_b), bn2_g, bn2_b))
    x = _maxpool2(x)
    x = jax.nn.relu(_bn2d(_conv(x, conv3_w, conv3_b), bn3_g, bn3_b))
    feat = x.reshape(x.shape[0], -1)
    feat = jax.nn.relu(_ln(feat @ fc1_w.T + fc1_b, ln_g, ln_b))
    dp = jax.nn.relu(_bn1d(feat @ disc_w.T + disc_b, bnd_g, bnd_b))
    sw = dp @ sw_w.T + sw_b
    u = jax.random.uniform(jax.random.key(1234), sw.shape, minval=1e-6,
                           maxval=1.0 - 1e-6)
    sg = sw - jnp.log(-jnp.log(u))

    w1cat = part_w1.reshape(512, 512)
    b1cat = part_b1.reshape(512, 1)
    w2cat = part_w2.transpose(1, 0, 2).reshape(1000, 512)
    b2t = part_b2.T

    g = _N // _NT
    cls, dom, idx, hard = pl.pallas_call(
        _head_kernel,
        grid=(g,),
        in_specs=[pl.BlockSpec((512, _NT), lambda i: (0, i)),
                  pl.BlockSpec((512, _NT), lambda i: (0, i)),
                  pl.BlockSpec((4, _NT), lambda i: (0, i)),
                  _full((2, 512)), _full((2, 1)),
                  _full((512, 512)), _full((512, 1)),
                  _full((1000, 512)), _full((1000, 4))],
        out_specs=[pl.BlockSpec((1000, _NT), lambda i: (0, i)),
                   pl.BlockSpec((2, _NT), lambda i: (0, i)),
                   pl.BlockSpec((1, _NT), lambda i: (0, i)),
                   pl.BlockSpec((4, _NT), lambda i: (0, i))],
        out_shape=[jax.ShapeDtypeStruct((1000, _N), F32),
                   jax.ShapeDtypeStruct((2, _N), F32),
                   jax.ShapeDtypeStruct((1, _N), jnp.int32),
                   jax.ShapeDtypeStruct((4, _N), F32)],
        compiler_params=_PARAMS,
    )(feat.T, dp.T, sg.T, discfc_w, discfc_b.reshape(2, 1), w1cat, b1cat,
      w2cat, b2t)

    return cls.T, dom.T, idx.reshape(_N), hard.T


# speedup vs baseline: 1.5778x; 1.0271x over previous
import jax
import jax.numpy as jnp
from jax.experimental import pallas as pl
from jax.experimental.pallas import tpu as pltpu

F32 = jnp.float32
_N = 1024
_NT = 128


def _head_kernel(feat_ref, dp_ref, sg_ref, dfw_ref, dfb_ref, w1_ref, b1_ref,
                 w2_ref, b2_ref, cls_ref, dom_ref, idx_ref, hard_ref):
    dp = dp_ref[...]
    dom_ref[...] = (jax.lax.dot_general(dp, dfw_ref[...],
                                        (((1,), (0,)), ((), ())),
                                        preferred_element_type=F32)
                    + dfb_ref[...])
    s = sg_ref[...]
    ii = jax.lax.broadcasted_iota(jnp.int32, s.shape, 1)
    mx = jnp.max(s, axis=1, keepdims=True)
    idx = jnp.min(jnp.where(s == mx, ii, s.shape[1]), axis=1, keepdims=True)
    hard = (ii == idx).astype(F32)
    idx_ref[...] = idx
    hard_ref[...] = hard
    h = jax.lax.dot_general(feat_ref[...], w1_ref[...],
                            (((1,), (0,)), ((), ())),
                            preferred_element_type=F32) + b1_ref[...]
    h = jnp.maximum(h, 0.0)
    mask = jnp.concatenate(
        [jnp.broadcast_to(hard[:, p:p + 1], (h.shape[0], 128))
         for p in range(4)], axis=1)
    h = h * mask
    b2 = jax.lax.dot_general(hard, b2_ref[...], (((1,), (0,)), ((), ())),
                             preferred_element_type=F32)
    cls_ref[...] = jax.lax.dot_general(h, w2_ref[...],
                                       (((1,), (0,)), ((), ())),
                                       preferred_element_type=F32) + b2


def _full(shape):
    nd = len(shape)
    return pl.BlockSpec(shape, lambda i: (0,) * nd)


_PARAMS = pltpu.CompilerParams(dimension_semantics=("arbitrary",))


def _bn2d(x, g, b, eps=1e-5):
    m = jnp.mean(x, axis=(0, 2, 3), keepdims=True)
    v = jnp.var(x, axis=(0, 2, 3), keepdims=True)
    return (x - m) / jnp.sqrt(v + eps) * g[None, :, None, None] + b[None, :, None, None]


def _bn1d(x, g, b, eps=1e-5):
    m = jnp.mean(x, axis=0, keepdims=True)
    v = jnp.var(x, axis=0, keepdims=True)
    return (x - m) / jnp.sqrt(v + eps) * g[None, :] + b[None, :]


def _ln(x, g, b, eps=1e-5):
    m = jnp.mean(x, axis=-1, keepdims=True)
    v = jnp.var(x, axis=-1, keepdims=True)
    return (x - m) / jnp.sqrt(v + eps) * g + b


def _conv(x, w, b):
    y = jax.lax.conv_general_dilated(x, w, window_strides=(1, 1),
                                     padding='SAME',
                                     dimension_numbers=('NCHW', 'OIHW', 'NCHW'))
    return y + b[None, :, None, None]


def _maxpool2(x):
    return jax.lax.reduce_window(x, -jnp.inf, jax.lax.max, (1, 1, 2, 2),
                                 (1, 1, 2, 2), 'VALID')


def kernel(input_data, conv1_w, conv1_b, bn1_g, bn1_b, conv2_w, conv2_b,
           bn2_g, bn2_b, conv3_w, conv3_b, bn3_g, bn3_b, fc1_w, fc1_b,
           ln_g, ln_b, disc_w, disc_b, bnd_g, bnd_b, discfc_w, discfc_b,
           sw_w, sw_b, part_w1, part_b1, part_w2, part_b2):
    x = jax.nn.relu(_bn2d(_conv(input_data, conv1_w, conv1_b), bn1_g, bn1_b))
    x = _maxpool2(x)
    x = jax.nn.relu(_bn2d(_conv(x, conv2_w, conv2_b), bn2_g, bn2_b))
    x = _maxpool2(x)
    x = jax.nn.relu(_bn2d(_conv(x, conv3_w, conv3_b), bn3_g, bn3_b))
    feat = x.reshape(x.shape[0], -1)
    feat = jax.nn.relu(_ln(feat @ fc1_w.T + fc1_b, ln_g, ln_b))
    dp = jax.nn.relu(_bn1d(feat @ disc_w.T + disc_b, bnd_g, bnd_b))
    sw = dp @ sw_w.T + sw_b
    u = jax.random.uniform(jax.random.key(1234), sw.shape, minval=1e-6,
                           maxval=1.0 - 1e-6)
    sg = sw - jnp.log(-jnp.log(u))

    w1catT = part_w1.reshape(512, 512).T
    b1cat = part_b1.reshape(1, 512)
    w2catT = part_w2.transpose(1, 0, 2).reshape(1000, 512).T

    g = _N // _NT
    cls, dom, idx, hard = pl.pallas_call(
        _head_kernel,
        grid=(g,),
        in_specs=[pl.BlockSpec((_NT, 512), lambda i: (i, 0)),
                  pl.BlockSpec((_NT, 512), lambda i: (i, 0)),
                  pl.BlockSpec((_NT, 4), lambda i: (i, 0)),
                  _full((512, 2)), _full((1, 2)),
                  _full((512, 512)), _full((1, 512)),
                  _full((512, 1000)), _full((4, 1000))],
        out_specs=[pl.BlockSpec((_NT, 1000), lambda i: (i, 0)),
                   pl.BlockSpec((_NT, 2), lambda i: (i, 0)),
                   pl.BlockSpec((_NT, 1), lambda i: (i, 0)),
                   pl.BlockSpec((_NT, 4), lambda i: (i, 0))],
        out_shape=[jax.ShapeDtypeStruct((_N, 1000), F32),
                   jax.ShapeDtypeStruct((_N, 2), F32),
                   jax.ShapeDtypeStruct((_N, 1), jnp.int32),
                   jax.ShapeDtypeStruct((_N, 4), F32)],
        compiler_params=_PARAMS,
    )(feat, dp, sg, discfc_w.T, discfc_b.reshape(1, 2), w1catT, b1cat,
      w2catT, part_b2)

    return cls, dom, idx.reshape(_N), hard
